# baseline (speedup 1.0000x reference)
import jax
import jax.numpy as jnp
from jax.experimental import pallas as pl
from jax.experimental.pallas import tpu as pltpu

N = 256
CH = 32
PACK = 4
G = 16


def _leaky(x):
    return jnp.where(x >= 0, x, 0.01 * x)


def _gnn_block(vt_ref, w1, bb1, w2, bb2, w3a, w3b, bb3, w4, bb4,
               w5, bb5, w6, bb6, w7, bb7, w8, bb8, s2, bgr, out_ref):
    R = N * G
    x = vt_ref[...].reshape(R, PACK * 11)

    def dot(a, w):
        return jax.lax.dot_general(a, w, (((1,), (0,)), ((), ())),
                                   preferred_element_type=jnp.float32)

    def roll_up(a):
        return jnp.concatenate([a[G:], a[:G]], axis=0)

    def roll_dn(a):
        return jnp.concatenate([a[R - G:], a[:R - G]], axis=0)

    def msgs_of(f):
        m = _leaky(dot(f, w3a[...]) + roll_up(dot(f, w3b[...])) + bb3[...])
        return _leaky(dot(m, w4[...]) + bb4[...])

    h = _leaky(dot(x, w1[...]) + bb1[...])
    v = _leaky(dot(h, w2[...]) + bb2[...])

    m = msgs_of(v)
    nv = 0.5 * (m + roll_dn(m))
    for _ in range(5):
        hh = _leaky(dot(nv, w5[...]) + bb5[...])
        hh = _leaky(dot(hh, w6[...]) + bb6[...])
        m = msgs_of(hh)
        nv = nv + 0.5 * (m + roll_dn(m))

    fo = _leaky(dot(nv, w7[...]) + bb7[...])
    fo = _leaky(dot(fo, w8[...]) + bb8[...])
    s = dot(s2[...], fo)
    out_ref[...] = jax.nn.sigmoid(s + bgr[...])


def kernel(vertices, W1, b1, W2, b2, W3, b3, W4, b4, W5, b5, W6, b6,
           W7, b7, W8, b8, Wg, bg, edges, dest_edges):
    B = vertices.shape[0]
    f32 = jnp.float32
    ngroups = B // PACK
    nblocks = ngroups // G
    R = N * G

    vt = jnp.transpose(vertices, (1, 0, 2)).reshape(N, ngroups, PACK * 11)

    eye = jnp.eye(PACK, dtype=f32)

    def bd(w):
        return jnp.kron(eye, w.T.astype(f32))

    def bt(b):
        return jnp.tile(b.astype(f32), PACK)[None, :]

    w1 = bd(W1)
    w2, w4, w5, w6, w7 = bd(W2), bd(W4), bd(W5), bd(W6), bd(W7)
    w3a, w3b = bd(W3[:, :CH]), bd(W3[:, CH:])
    w8 = bd(W8)
    bb1, bb2, bb3, bb4 = bt(b1), bt(b2), bt(b3), bt(b4)
    bb5, bb6, bb7, bb8 = bt(b5), bt(b6), bt(b7), bt(b8)

    r = jnp.arange(R)
    s2 = ((r[None, :] % G) == jnp.arange(G)[:, None]).astype(f32) \
        * Wg[0, r // G][None, :].astype(f32)
    bgr = jnp.broadcast_to(bg.astype(f32).reshape(1, 1), (1, 1))

    full = lambda a: pl.BlockSpec(a.shape, lambda i: (0,) * a.ndim)
    ws = [w1, bb1, w2, bb2, w3a, w3b, bb3, w4, bb4,
          w5, bb5, w6, bb6, w7, bb7, w8, bb8, s2, bgr]

    out = pl.pallas_call(
        _gnn_block,
        grid=(nblocks,),
        in_specs=[pl.BlockSpec((N, G, PACK * 11), lambda i: (0, i, 0))]
                 + [full(a) for a in ws],
        out_specs=pl.BlockSpec((G, PACK), lambda i: (i, 0)),
        out_shape=jax.ShapeDtypeStruct((ngroups, PACK), f32),
        compiler_params=pltpu.CompilerParams(
            dimension_semantics=("parallel",)),
    )(vt, *ws)
    return out.reshape(B, 1)


# speedup vs baseline: 1.0338x; 1.0338x over previous
import jax
import jax.numpy as jnp
from jax.experimental import pallas as pl
from jax.experimental.pallas import tpu as pltpu

N = 256
CH = 32
PACK = 4
G = 8


def _leaky(x):
    return jnp.where(x >= 0, x, 0.01 * x)


def _gnn_block(vt_ref, w1, bb1, w2, bb2, w3a, w3b, bb3, w4, bb4,
               w5, bb5, w6, bb6, w7, bb7, w8, bb8, s2, bgr, out_ref):
    R = N * G
    x = vt_ref[...].reshape(R, PACK * 11)

    def dot(a, w):
        return jax.lax.dot_general(a, w, (((1,), (0,)), ((), ())),
                                   preferred_element_type=jnp.float32)

    def roll_up(a):
        return jnp.concatenate([a[G:], a[:G]], axis=0)

    def roll_dn(a):
        return jnp.concatenate([a[R - G:], a[:R - G]], axis=0)

    def msgs_of(f):
        m = _leaky(dot(f, w3a[...]) + roll_up(dot(f, w3b[...])) + bb3[...])
        return _leaky(dot(m, w4[...]) + bb4[...])

    h = _leaky(dot(x, w1[...]) + bb1[...])
    v = _leaky(dot(h, w2[...]) + bb2[...])

    m = msgs_of(v)
    nv = 0.5 * (m + roll_dn(m))
    for _ in range(5):
        hh = _leaky(dot(nv, w5[...]) + bb5[...])
        hh = _leaky(dot(hh, w6[...]) + bb6[...])
        m = msgs_of(hh)
        nv = nv + 0.5 * (m + roll_dn(m))

    fo = _leaky(dot(nv, w7[...]) + bb7[...])
    fo = _leaky(dot(fo, w8[...]) + bb8[...])
    s = dot(s2[...], fo)
    out_ref[...] = jax.nn.sigmoid(s + bgr[...])


def kernel(vertices, W1, b1, W2, b2, W3, b3, W4, b4, W5, b5, W6, b6,
           W7, b7, W8, b8, Wg, bg, edges, dest_edges):
    B = vertices.shape[0]
    f32 = jnp.float32
    ngroups = B // PACK
    nblocks = ngroups // G
    R = N * G

    vt = jnp.transpose(vertices, (1, 0, 2)).reshape(N, ngroups, PACK * 11)

    eye = jnp.eye(PACK, dtype=f32)

    def bd(w):
        return jnp.kron(eye, w.T.astype(f32))

    def bt(b):
        return jnp.tile(b.astype(f32), PACK)[None, :]

    w1 = bd(W1)
    w2, w4, w5, w6, w7 = bd(W2), bd(W4), bd(W5), bd(W6), bd(W7)
    w3a, w3b = bd(W3[:, :CH]), bd(W3[:, CH:])
    w8 = bd(W8)
    bb1, bb2, bb3, bb4 = bt(b1), bt(b2), bt(b3), bt(b4)
    bb5, bb6, bb7, bb8 = bt(b5), bt(b6), bt(b7), bt(b8)

    r = jnp.arange(R)
    s2 = ((r[None, :] % G) == jnp.arange(G)[:, None]).astype(f32) \
        * Wg[0, r // G][None, :].astype(f32)
    bgr = jnp.broadcast_to(bg.astype(f32).reshape(1, 1), (1, 1))

    full = lambda a: pl.BlockSpec(a.shape, lambda i: (0,) * a.ndim)
    ws = [w1, bb1, w2, bb2, w3a, w3b, bb3, w4, bb4,
          w5, bb5, w6, bb6, w7, bb7, w8, bb8, s2, bgr]

    out = pl.pallas_call(
        _gnn_block,
        grid=(nblocks,),
        in_specs=[pl.BlockSpec((N, G, PACK * 11), lambda i: (0, i, 0))]
                 + [full(a) for a in ws],
        out_specs=pl.BlockSpec((G, PACK), lambda i: (i, 0)),
        out_shape=jax.ShapeDtypeStruct((ngroups, PACK), f32),
        compiler_params=pltpu.CompilerParams(
            dimension_semantics=("parallel",)),
    )(vt, *ws)
    return out.reshape(B, 1)
